# baseline (speedup 1.0000x reference)
import functools

import jax
import jax.numpy as jnp
import numpy as np
from jax import lax
from jax.experimental import pallas as pl
from jax.experimental.pallas import tpu as pltpu
from jax.experimental.pallas import tpu_sc as plsc

B = 32
V = 1_000_000

CHUNK = 8192
NCHUNK = 97
C0 = NCHUNK * CHUNK
SC_COLS = V - C0
SC_CH = 49152
SC_SIZES = [SC_CH] * (SC_COLS // SC_CH) + (
    [SC_COLS % SC_CH] if SC_COLS % SC_CH else []
)
TILE = 512
NTILE = CHUNK // TILE

SC_LANES = 16
SC_UNROLL = 4

K1 = 0
K2 = 42
KS2 = K1 ^ K2 ^ 0x1BD11BDA

_ROT_A = (13, 15, 26, 6)
_ROT_B = (17, 29, 16, 24)

_TINY = np.float32(np.finfo(np.float32).tiny)
_NEG_INF = np.float32(-np.inf)
_LN2 = np.float32(0.6931471805599453)
_LNR_COEF = (
    4.054651051e-01, 3.333333387e-01, -5.555539352e-02, 1.234556493e-02,
    -3.087803102e-03, 8.237186327e-04, -2.245755447e-04, 6.372952435e-05,
    -2.372999076e-05, 7.236465312e-06,
)


def _rotl(x, r):
    return lax.shift_left(x, jnp.uint32(r)) | lax.shift_right_logical(
        x, jnp.uint32(32 - r)
    )


def _rounds(x0, x1, rots):
    for r in rots:
        x0 = x0 + x1
        x1 = _rotl(x1, r)
        x1 = x0 ^ x1
    return x0, x1


def _threefry_bits(i):
    ks0 = jnp.uint32(K1)
    ks1 = jnp.uint32(K2)
    ks2 = jnp.uint32(KS2)
    x0 = jnp.full_like(i, ks0)
    x1 = i + ks1
    x0, x1 = _rounds(x0, x1, _ROT_A)
    x0 = x0 + ks1
    x1 = x1 + ks2 + jnp.uint32(1)
    x0, x1 = _rounds(x0, x1, _ROT_B)
    x0 = x0 + ks2
    x1 = x1 + ks0 + jnp.uint32(2)
    x0, x1 = _rounds(x0, x1, _ROT_A)
    x0 = x0 + ks0
    x1 = x1 + ks1 + jnp.uint32(3)
    x0, x1 = _rounds(x0, x1, _ROT_B)
    x0 = x0 + ks1
    x1 = x1 + ks2 + jnp.uint32(4)
    x0, x1 = _rounds(x0, x1, _ROT_A)
    x0 = x0 + ks2
    x1 = x1 + ks0 + jnp.uint32(5)
    return x0 ^ x1


def _uniform_from_bits(bits):
    fbits = lax.shift_right_logical(bits, jnp.uint32(9)) | jnp.uint32(0x3F800000)
    f = lax.bitcast_convert_type(fbits, jnp.float32) - jnp.float32(1.0)
    return jnp.maximum(_TINY, f + _TINY)


def _score(bits, x):
    u = _uniform_from_bits(bits)
    w = jnp.float32(0.0) - jnp.log(u)
    return x - jnp.log(w)


def _tc_kernel(x_ref, val_ref, idx_ref, acc_val, acc_idx):
    pid = pl.program_id(0)

    base = jax.lax.broadcasted_iota(jnp.uint32, (B, TILE), 0) * jnp.uint32(V) + (
        jax.lax.broadcasted_iota(jnp.uint32, (B, TILE), 1)
        + pid.astype(jnp.uint32) * jnp.uint32(CHUNK)
    )
    col_base = jax.lax.broadcasted_iota(jnp.int32, (B, TILE), 1)
    col0 = pid * jnp.int32(CHUNK)

    def tile_step(t):
        flat = base + jnp.uint32(t * TILE)
        bits = _threefry_bits(flat)
        v = _score(bits, x_ref[:, pl.ds(t * TILE, TILE)])
        col = col_base + (col0 + jnp.int32(t * TILE))
        better = v > acc_val[...]
        acc_val[...] = jnp.where(better, v, acc_val[...])
        acc_idx[...] = jnp.where(better, col, acc_idx[...])

    @pl.when(pid == 0)
    def _first_chunk():
        bits = _threefry_bits(base)
        acc_val[...] = _score(bits, x_ref[:, pl.ds(0, TILE)])
        acc_idx[...] = col_base
        for t in range(1, NTILE):
            tile_step(t)

    @pl.when(pid != 0)
    def _rest():
        for t in range(NTILE):
            tile_step(t)

    @pl.when(pid == NCHUNK - 1)
    def _finish():
        av = acc_val[...]
        ai = acc_idx[...]
        m = jnp.max(av, axis=1, keepdims=True)
        cand = jnp.where(av == m, ai, jnp.int32(2**31 - 1))
        val_ref[0, :] = jnp.max(av, axis=1)
        idx_ref[0, :] = jnp.min(cand, axis=1)


def _tc_partial(logits):
    return pl.pallas_call(
        _tc_kernel,
        grid=(NCHUNK,),
        in_specs=[pl.BlockSpec((B, CHUNK), lambda c: (0, c))],
        out_specs=(
            pl.BlockSpec((1, B), lambda c: (0, 0)),
            pl.BlockSpec((1, B), lambda c: (0, 0)),
        ),
        out_shape=(
            jax.ShapeDtypeStruct((1, B), jnp.float32),
            jax.ShapeDtypeStruct((1, B), jnp.int32),
        ),
        scratch_shapes=[
            pltpu.VMEM((B, TILE), jnp.float32),
            pltpu.VMEM((B, TILE), jnp.int32),
        ],
        compiler_params=pltpu.CompilerParams(
            dimension_semantics=("arbitrary",),
        ),
    )(logits)


def _sc_key(bits, x):
    u = _uniform_from_bits(bits)
    d = jnp.float32(1.0) - u
    pa = jnp.float32(1.0 / 12.0)
    for n in range(11, 0, -1):
        pa = jnp.float32(1.0 / n) + d * pa
    w_a = d * pa
    ub = lax.bitcast_convert_type(u, jnp.uint32)
    e = (lax.shift_right_logical(ub, jnp.uint32(23))).astype(jnp.int32) - 127
    r = lax.bitcast_convert_type(
        (ub & jnp.uint32(0x7FFFFF)) | jnp.uint32(0x3F800000), jnp.float32
    )
    s = (r - jnp.float32(1.0)) / (r + jnp.float32(1.0))
    s2 = s * s
    ln_r = s * (
        jnp.float32(2.0)
        + s2
        * (
            jnp.float32(2.0 / 3.0)
            + s2
            * (
                jnp.float32(2.0 / 5.0)
                + s2 * (jnp.float32(2.0 / 7.0) + s2 * jnp.float32(2.0 / 9.0))
            )
        )
    )
    w_b = e.astype(jnp.float32) * (-_LN2) - ln_r
    w = jnp.where(u >= jnp.float32(0.75), w_a, w_b)
    return w * jnp.exp(jnp.float32(0.0) - x)


def _sc_body(logits_hbm, col_out, x_out, buf0, buf1, buf_tail, col_v, xv_v, sem0, sem1):
    wid = lax.axis_index("s") * 2 + lax.axis_index("c")
    row = wid

    lane = lax.iota(jnp.int32, 16)
    row_flat = row * jnp.int32(V)
    bufs = (buf0, buf1)
    sems = (sem0, sem1)

    def chunk_buf(j):
        return bufs[j % 2] if SC_SIZES[j] == SC_CH else buf_tail

    def start(j):
        size = SC_SIZES[j]
        src = logits_hbm.at[row, pl.ds(C0 + j * SC_CH, size)]
        desc = pltpu.make_async_copy(src, chunk_buf(j), sems[j % 2])
        desc.start()
        return desc

    def process(buf, size, cb, carry):
        unroll = SC_UNROLL if size % (SC_UNROLL * SC_LANES) == 0 else 4

        def step(k, carry):
            tmin, cmin, xmin = carry
            for j in range(unroll):
                off = k * (unroll * SC_LANES) + jnp.int32(j * SC_LANES)
                x = buf[pl.ds(off, SC_LANES)]
                col = lane + (jnp.int32(cb) + off)
                flat = (row_flat + col).astype(jnp.uint32)
                bits = _threefry_bits(flat)
                t = _sc_key(bits, x)
                better = t < tmin
                tmin = jnp.where(better, t, tmin)
                cmin = jnp.where(better, col, cmin)
                xmin = jnp.where(better, x, xmin)
            return tmin, cmin, xmin

        return lax.fori_loop(0, size // (unroll * SC_LANES), step, carry)

    carry = (
        jnp.full((SC_LANES,), jnp.inf, jnp.float32),
        jnp.zeros((SC_LANES,), jnp.int32),
        jnp.zeros((SC_LANES,), jnp.float32),
    )
    h_cur = start(0)
    for j in range(len(SC_SIZES)):
        h_next = start(j + 1) if j + 1 < len(SC_SIZES) else None
        h_cur.wait()
        carry = process(chunk_buf(j), SC_SIZES[j], C0 + j * SC_CH, carry)
        h_cur = h_next

    _, cmin, xmin = carry
    col_v[...] = cmin
    xv_v[...] = xmin
    pltpu.sync_copy(col_v, col_out.at[row])
    pltpu.sync_copy(xv_v, x_out.at[row])


@functools.cache
def _sc_tail_kernel():
    return pl.kernel(
        _sc_body,
        out_type=(
            jax.ShapeDtypeStruct((B, SC_LANES), jnp.int32),
            jax.ShapeDtypeStruct((B, SC_LANES), jnp.float32),
        ),
        mesh=plsc.VectorSubcoreMesh(core_axis_name="c", subcore_axis_name="s"),
        scratch_types=[
            pltpu.VMEM((SC_CH,), jnp.float32),
            pltpu.VMEM((SC_CH,), jnp.float32),
            pltpu.VMEM((SC_SIZES[-1],), jnp.float32),
            pltpu.VMEM((SC_LANES,), jnp.int32),
            pltpu.VMEM((SC_LANES,), jnp.float32),
            pltpu.SemaphoreType.DMA,
            pltpu.SemaphoreType.DMA,
        ],
    )


def _merge_kernel(val_ref, idx_ref, sccol_ref, scx_ref, o_ref):
    col = sccol_ref[...]
    x = scx_ref[...]
    row = jax.lax.broadcasted_iota(jnp.uint32, (B, SC_LANES), 0)
    flat = row * jnp.uint32(V) + col.astype(jnp.uint32)
    s = _score(_threefry_bits(flat), x)
    m = jnp.max(s, axis=1, keepdims=True)
    cand = jnp.where(s == m, col, jnp.int32(2**31 - 1))
    sc_col = jnp.min(cand, axis=1)
    sc_val = m[:, 0]
    take_sc = sc_val > val_ref[0, :]
    o_ref[0, :] = jnp.where(take_sc, sc_col, idx_ref[0, :])


def _merge(tc_val, tc_idx, sc_col, sc_x):
    return pl.pallas_call(
        _merge_kernel,
        out_shape=jax.ShapeDtypeStruct((1, B), jnp.int32),
    )(tc_val, tc_idx, sc_col, sc_x)


@jax.jit
def kernel(logits):
    sc_col, sc_x = _sc_tail_kernel()(logits)
    tc_val, tc_idx = _tc_partial(logits)
    row = jax.lax.broadcasted_iota(jnp.uint32, (B, SC_LANES), 0)
    flat = row * jnp.uint32(V) + sc_col.astype(jnp.uint32)
    s = _score(_threefry_bits(flat), sc_x)
    m = jnp.max(s, axis=1, keepdims=True)
    cand = jnp.where(s == m, sc_col, jnp.int32(2**31 - 1))
    sc_best_col = jnp.min(cand, axis=1)
    take_sc = m[:, 0] > tc_val[0, :]
    return jnp.where(take_sc, sc_best_col, tc_idx[0, :])


# speedup vs baseline: 1.0030x; 1.0030x over previous
import functools

import jax
import jax.numpy as jnp
import numpy as np
from jax import lax
from jax.experimental import pallas as pl
from jax.experimental.pallas import tpu as pltpu
from jax.experimental.pallas import tpu_sc as plsc

B = 32
V = 1_000_000

CHUNK = 8192
NCHUNK = 97
C0 = NCHUNK * CHUNK
SC_COLS = V - C0
SC_CH = 49152
SC_SIZES = [SC_CH] * (SC_COLS // SC_CH) + (
    [SC_COLS % SC_CH] if SC_COLS % SC_CH else []
)
TILE = 512
NTILE = CHUNK // TILE

SC_LANES = 16
SC_UNROLL = 4

K1 = 0
K2 = 42
KS2 = K1 ^ K2 ^ 0x1BD11BDA

_ROT_A = (13, 15, 26, 6)
_ROT_B = (17, 29, 16, 24)

_TINY = np.float32(np.finfo(np.float32).tiny)
_NEG_INF = np.float32(-np.inf)
_LN2 = np.float32(0.6931471805599453)
_LNR_COEF = (
    4.054651051e-01, 3.333333387e-01, -5.555539352e-02, 1.234556493e-02,
    -3.087803102e-03, 8.237186327e-04, -2.245755447e-04, 6.372952435e-05,
    -2.372999076e-05, 7.236465312e-06,
)


def _rotl(x, r):
    return lax.shift_left(x, jnp.uint32(r)) | lax.shift_right_logical(
        x, jnp.uint32(32 - r)
    )


def _rounds(x0, x1, rots):
    for r in rots:
        x0 = x0 + x1
        x1 = _rotl(x1, r)
        x1 = x0 ^ x1
    return x0, x1


def _threefry_bits(i):
    ks0 = jnp.uint32(K1)
    ks1 = jnp.uint32(K2)
    ks2 = jnp.uint32(KS2)
    x0 = jnp.full_like(i, ks0)
    x1 = i + ks1
    x0, x1 = _rounds(x0, x1, _ROT_A)
    x0 = x0 + ks1
    x1 = x1 + ks2 + jnp.uint32(1)
    x0, x1 = _rounds(x0, x1, _ROT_B)
    x0 = x0 + ks2
    x1 = x1 + ks0 + jnp.uint32(2)
    x0, x1 = _rounds(x0, x1, _ROT_A)
    x0 = x0 + ks0
    x1 = x1 + ks1 + jnp.uint32(3)
    x0, x1 = _rounds(x0, x1, _ROT_B)
    x0 = x0 + ks1
    x1 = x1 + ks2 + jnp.uint32(4)
    x0, x1 = _rounds(x0, x1, _ROT_A)
    x0 = x0 + ks2
    x1 = x1 + ks0 + jnp.uint32(5)
    return x0 ^ x1


def _uniform_from_bits(bits):
    fbits = lax.shift_right_logical(bits, jnp.uint32(9)) | jnp.uint32(0x3F800000)
    f = lax.bitcast_convert_type(fbits, jnp.float32) - jnp.float32(1.0)
    return jnp.maximum(_TINY, f + _TINY)


def _score(bits, x):
    u = _uniform_from_bits(bits)
    w = jnp.float32(0.0) - jnp.log(u)
    return x - jnp.log(w)


def _tc_kernel(x_ref, val_ref, idx_ref, acc_val, acc_idx):
    pid = pl.program_id(0)

    base = jax.lax.broadcasted_iota(jnp.uint32, (B, TILE), 0) * jnp.uint32(V) + (
        jax.lax.broadcasted_iota(jnp.uint32, (B, TILE), 1)
        + pid.astype(jnp.uint32) * jnp.uint32(CHUNK)
    )
    col_base = jax.lax.broadcasted_iota(jnp.int32, (B, TILE), 1)
    col0 = pid * jnp.int32(CHUNK)

    def tile_step(t):
        flat = base + jnp.uint32(t * TILE)
        bits = _threefry_bits(flat)
        v = _score(bits, x_ref[:, pl.ds(t * TILE, TILE)])
        col = col_base + (col0 + jnp.int32(t * TILE))
        better = v > acc_val[...]
        acc_val[...] = jnp.where(better, v, acc_val[...])
        acc_idx[...] = jnp.where(better, col, acc_idx[...])

    @pl.when(pid == 0)
    def _first_chunk():
        bits = _threefry_bits(base)
        acc_val[...] = _score(bits, x_ref[:, pl.ds(0, TILE)])
        acc_idx[...] = col_base
        for t in range(1, NTILE):
            tile_step(t)

    @pl.when(pid != 0)
    def _rest():
        for t in range(NTILE):
            tile_step(t)

    @pl.when(pid == NCHUNK - 1)
    def _finish():
        av = acc_val[...]
        ai = acc_idx[...]
        m = jnp.max(av, axis=1, keepdims=True)
        cand = jnp.where(av == m, ai, jnp.int32(2**31 - 1))
        val_ref[0, :] = jnp.max(av, axis=1)
        idx_ref[0, :] = jnp.min(cand, axis=1)


def _tc_partial(logits):
    return pl.pallas_call(
        _tc_kernel,
        grid=(NCHUNK,),
        in_specs=[pl.BlockSpec((B, CHUNK), lambda c: (0, c))],
        out_specs=(
            pl.BlockSpec((1, B), lambda c: (0, 0)),
            pl.BlockSpec((1, B), lambda c: (0, 0)),
        ),
        out_shape=(
            jax.ShapeDtypeStruct((1, B), jnp.float32),
            jax.ShapeDtypeStruct((1, B), jnp.int32),
        ),
        scratch_shapes=[
            pltpu.VMEM((B, TILE), jnp.float32),
            pltpu.VMEM((B, TILE), jnp.int32),
        ],
        compiler_params=pltpu.CompilerParams(
            dimension_semantics=("arbitrary",),
        ),
    )(logits)


def _sc_key(bits, x):
    u = _uniform_from_bits(bits)
    d = jnp.float32(1.0) - u
    pa = jnp.float32(1.0 / 12.0)
    for n in range(11, 0, -1):
        pa = jnp.float32(1.0 / n) + d * pa
    w_a = d * pa
    ub = lax.bitcast_convert_type(u, jnp.uint32)
    e = (lax.shift_right_logical(ub, jnp.uint32(23))).astype(jnp.int32) - 127
    r = lax.bitcast_convert_type(
        (ub & jnp.uint32(0x7FFFFF)) | jnp.uint32(0x3F800000), jnp.float32
    )
    s = (r - jnp.float32(1.0)) / (r + jnp.float32(1.0))
    s2 = s * s
    ln_r = s * (
        jnp.float32(2.0)
        + s2
        * (
            jnp.float32(2.0 / 3.0)
            + s2
            * (
                jnp.float32(2.0 / 5.0)
                + s2 * (jnp.float32(2.0 / 7.0) + s2 * jnp.float32(2.0 / 9.0))
            )
        )
    )
    w_b = e.astype(jnp.float32) * (-_LN2) - ln_r
    w = jnp.where(u >= jnp.float32(0.75), w_a, w_b)
    return w * jnp.exp(jnp.float32(0.0) - x)


def _sc_body(logits_hbm, col_out, x_out, buf0, buf1, buf_tail, col_v, xv_v, sem0, sem1):
    wid = lax.axis_index("s") * 2 + lax.axis_index("c")
    row = wid

    lane = lax.iota(jnp.int32, 16)
    row_flat = row * jnp.int32(V)
    bufs = (buf0, buf1)
    sems = (sem0, sem1)

    def chunk_buf(j):
        return bufs[j % 2] if SC_SIZES[j] == SC_CH else buf_tail

    def start(j):
        size = SC_SIZES[j]
        src = logits_hbm.at[row, pl.ds(C0 + j * SC_CH, size)]
        desc = pltpu.make_async_copy(src, chunk_buf(j), sems[j % 2])
        desc.start()
        return desc

    def process(buf, size, cb, carry):
        unroll = SC_UNROLL if size % (SC_UNROLL * SC_LANES) == 0 else 4

        def step(k, carry):
            tmin, cmin, xmin = carry
            for j in range(unroll):
                off = k * (unroll * SC_LANES) + jnp.int32(j * SC_LANES)
                x = buf[pl.ds(off, SC_LANES)]
                col = lane + (jnp.int32(cb) + off)
                flat = (row_flat + col).astype(jnp.uint32)
                bits = _threefry_bits(flat)
                t = _sc_key(bits, x)
                better = t < tmin
                tmin = jnp.where(better, t, tmin)
                cmin = jnp.where(better, col, cmin)
                xmin = jnp.where(better, x, xmin)
            return tmin, cmin, xmin

        return lax.fori_loop(0, size // (unroll * SC_LANES), step, carry)

    carry = (
        jnp.full((SC_LANES,), jnp.inf, jnp.float32),
        jnp.zeros((SC_LANES,), jnp.int32),
        jnp.zeros((SC_LANES,), jnp.float32),
    )
    h_cur = start(0)
    for j in range(len(SC_SIZES)):
        h_next = start(j + 1) if j + 1 < len(SC_SIZES) else None
        h_cur.wait()
        carry = process(chunk_buf(j), SC_SIZES[j], C0 + j * SC_CH, carry)
        h_cur = h_next

    _, cmin, xmin = carry
    col_v[...] = cmin
    xv_v[...] = xmin
    pltpu.sync_copy(col_v, col_out.at[row])
    pltpu.sync_copy(xv_v, x_out.at[row])


@functools.cache
def _sc_tail_kernel():
    return pl.kernel(
        _sc_body,
        out_type=(
            jax.ShapeDtypeStruct((B, SC_LANES), jnp.int32),
            jax.ShapeDtypeStruct((B, SC_LANES), jnp.float32),
        ),
        mesh=plsc.VectorSubcoreMesh(core_axis_name="c", subcore_axis_name="s"),
        scratch_types=[
            pltpu.VMEM((SC_CH,), jnp.float32),
            pltpu.VMEM((SC_CH,), jnp.float32),
            pltpu.VMEM((SC_SIZES[-1],), jnp.float32),
            pltpu.VMEM((SC_LANES,), jnp.int32),
            pltpu.VMEM((SC_LANES,), jnp.float32),
            pltpu.SemaphoreType.DMA,
            pltpu.SemaphoreType.DMA,
        ],
    )


def _merge_kernel(val_ref, idx_ref, sccol_ref, scx_ref, o_ref):
    col = sccol_ref[...]
    x = scx_ref[...]
    row = jax.lax.broadcasted_iota(jnp.uint32, (B, SC_LANES), 0)
    flat = row * jnp.uint32(V) + col.astype(jnp.uint32)
    s = _score(_threefry_bits(flat), x)
    m = jnp.max(s, axis=1, keepdims=True)
    cand = jnp.where(s == m, col, jnp.int32(2**31 - 1))
    sc_col = jnp.min(cand, axis=1)
    sc_val = m[:, 0]
    take_sc = sc_val > val_ref[0, :]
    o_ref[0, :] = jnp.where(take_sc, sc_col, idx_ref[0, :])


def _merge(tc_val, tc_idx, sc_col, sc_x):
    return pl.pallas_call(
        _merge_kernel,
        out_shape=jax.ShapeDtypeStruct((1, B), jnp.int32),
    )(tc_val, tc_idx, sc_col, sc_x)


@jax.jit
def kernel(logits):
    sc_col, sc_x = _sc_tail_kernel()(logits)
    tc_val, tc_idx = _tc_partial(logits)
    out = _merge(tc_val, tc_idx, sc_col, sc_x)
    return out[0]


# speedup vs baseline: 1.0033x; 1.0003x over previous
import functools

import jax
import jax.numpy as jnp
import numpy as np
from jax import lax
from jax.experimental import pallas as pl
from jax.experimental.pallas import tpu as pltpu
from jax.experimental.pallas import tpu_sc as plsc

B = 32
V = 1_000_000

CHUNK = 8192
NCHUNK = 97
C0 = NCHUNK * CHUNK
SC_COLS = V - C0
SC_CH = 49152
SC_SIZES = [SC_CH] * (SC_COLS // SC_CH) + (
    [SC_COLS % SC_CH] if SC_COLS % SC_CH else []
)
TILE = 512
NTILE = CHUNK // TILE

SC_LANES = 16
SC_UNROLL = 4

K1 = 0
K2 = 42
KS2 = K1 ^ K2 ^ 0x1BD11BDA

_ROT_A = (13, 15, 26, 6)
_ROT_B = (17, 29, 16, 24)

_TINY = np.float32(np.finfo(np.float32).tiny)
_LN2 = np.float32(0.6931471805599453)


def _rotl(x, r):
    return lax.shift_left(x, jnp.uint32(r)) | lax.shift_right_logical(
        x, jnp.uint32(32 - r)
    )


def _rounds(x0, x1, rots):
    for r in rots:
        x0 = x0 + x1
        x1 = _rotl(x1, r)
        x1 = x0 ^ x1
    return x0, x1


def _threefry_bits(i):
    ks0 = jnp.uint32(K1)
    ks1 = jnp.uint32(K2)
    ks2 = jnp.uint32(KS2)
    x0 = jnp.full_like(i, ks0)
    x1 = i + ks1
    x0, x1 = _rounds(x0, x1, _ROT_A)
    x0 = x0 + ks1
    x1 = x1 + ks2 + jnp.uint32(1)
    x0, x1 = _rounds(x0, x1, _ROT_B)
    x0 = x0 + ks2
    x1 = x1 + ks0 + jnp.uint32(2)
    x0, x1 = _rounds(x0, x1, _ROT_A)
    x0 = x0 + ks0
    x1 = x1 + ks1 + jnp.uint32(3)
    x0, x1 = _rounds(x0, x1, _ROT_B)
    x0 = x0 + ks1
    x1 = x1 + ks2 + jnp.uint32(4)
    x0, x1 = _rounds(x0, x1, _ROT_A)
    x0 = x0 + ks2
    x1 = x1 + ks0 + jnp.uint32(5)
    return x0 ^ x1


def _uniform_from_bits(bits):
    fbits = lax.shift_right_logical(bits, jnp.uint32(9)) | jnp.uint32(0x3F800000)
    f = lax.bitcast_convert_type(fbits, jnp.float32) - jnp.float32(1.0)
    return jnp.maximum(_TINY, f + _TINY)


def _score(bits, x):
    u = _uniform_from_bits(bits)
    w = jnp.float32(0.0) - jnp.log(u)
    return x - jnp.log(w)


def _tc_kernel(x_ref, val_ref, idx_ref, acc_val, acc_idx):
    pid = pl.program_id(0)

    base = jax.lax.broadcasted_iota(jnp.uint32, (B, TILE), 0) * jnp.uint32(V) + (
        jax.lax.broadcasted_iota(jnp.uint32, (B, TILE), 1)
        + pid.astype(jnp.uint32) * jnp.uint32(CHUNK)
    )
    col_base = jax.lax.broadcasted_iota(jnp.int32, (B, TILE), 1)
    col0 = pid * jnp.int32(CHUNK)

    def tile_step(t):
        flat = base + jnp.uint32(t * TILE)
        bits = _threefry_bits(flat)
        v = _score(bits, x_ref[:, pl.ds(t * TILE, TILE)])
        col = col_base + (col0 + jnp.int32(t * TILE))
        better = v > acc_val[...]
        acc_val[...] = jnp.where(better, v, acc_val[...])
        acc_idx[...] = jnp.where(better, col, acc_idx[...])

    @pl.when(pid == 0)
    def _first_chunk():
        bits = _threefry_bits(base)
        acc_val[...] = _score(bits, x_ref[:, pl.ds(0, TILE)])
        acc_idx[...] = col_base
        for t in range(1, NTILE):
            tile_step(t)

    @pl.when(pid != 0)
    def _rest():
        for t in range(NTILE):
            tile_step(t)

    @pl.when(pid == NCHUNK - 1)
    def _finish():
        av = acc_val[...]
        ai = acc_idx[...]
        m = jnp.max(av, axis=1, keepdims=True)
        cand = jnp.where(av == m, ai, jnp.int32(2**31 - 1))
        val_ref[0, :] = jnp.max(av, axis=1)
        idx_ref[0, :] = jnp.min(cand, axis=1)


def _tc_partial(logits):
    return pl.pallas_call(
        _tc_kernel,
        grid=(NCHUNK,),
        in_specs=[pl.BlockSpec((B, CHUNK), lambda c: (0, c))],
        out_specs=(
            pl.BlockSpec((1, B), lambda c: (0, 0)),
            pl.BlockSpec((1, B), lambda c: (0, 0)),
        ),
        out_shape=(
            jax.ShapeDtypeStruct((1, B), jnp.float32),
            jax.ShapeDtypeStruct((1, B), jnp.int32),
        ),
        scratch_shapes=[
            pltpu.VMEM((B, TILE), jnp.float32),
            pltpu.VMEM((B, TILE), jnp.int32),
        ],
        compiler_params=pltpu.CompilerParams(
            dimension_semantics=("arbitrary",),
        ),
    )(logits)


def _sc_key(bits, x):
    u = _uniform_from_bits(bits)
    d = jnp.float32(1.0) - u
    pa = jnp.float32(1.0 / 12.0)
    for n in range(11, 0, -1):
        pa = jnp.float32(1.0 / n) + d * pa
    w_a = d * pa
    ub = lax.bitcast_convert_type(u, jnp.uint32)
    e = (lax.shift_right_logical(ub, jnp.uint32(23))).astype(jnp.int32) - 127
    r = lax.bitcast_convert_type(
        (ub & jnp.uint32(0x7FFFFF)) | jnp.uint32(0x3F800000), jnp.float32
    )
    s = (r - jnp.float32(1.0)) / (r + jnp.float32(1.0))
    s2 = s * s
    ln_r = s * (
        jnp.float32(2.0)
        + s2
        * (
            jnp.float32(2.0 / 3.0)
            + s2
            * (
                jnp.float32(2.0 / 5.0)
                + s2 * (jnp.float32(2.0 / 7.0) + s2 * jnp.float32(2.0 / 9.0))
            )
        )
    )
    w_b = e.astype(jnp.float32) * (-_LN2) - ln_r
    w = jnp.where(u >= jnp.float32(0.75), w_a, w_b)
    return w * jnp.exp(jnp.float32(0.0) - x)


def _sc_body(logits_hbm, col_out, x_out, buf0, buf1, buf_tail, col_v, xv_v, sem0, sem1):
    wid = lax.axis_index("s") * 2 + lax.axis_index("c")
    row = wid

    lane = lax.iota(jnp.int32, 16)
    row_flat = row * jnp.int32(V)
    bufs = (buf0, buf1)
    sems = (sem0, sem1)

    def chunk_buf(j):
        return bufs[j % 2] if SC_SIZES[j] == SC_CH else buf_tail

    def start(j):
        size = SC_SIZES[j]
        src = logits_hbm.at[row, pl.ds(C0 + j * SC_CH, size)]
        desc = pltpu.make_async_copy(src, chunk_buf(j), sems[j % 2])
        desc.start()
        return desc

    def process(buf, size, cb, carry):
        unroll = SC_UNROLL if size % (SC_UNROLL * SC_LANES) == 0 else 4

        def step(k, carry):
            tmin, cmin, xmin = carry
            for j in range(unroll):
                off = k * (unroll * SC_LANES) + jnp.int32(j * SC_LANES)
                x = buf[pl.ds(off, SC_LANES)]
                col = lane + (jnp.int32(cb) + off)
                flat = (row_flat + col).astype(jnp.uint32)
                bits = _threefry_bits(flat)
                t = _sc_key(bits, x)
                better = t < tmin
                tmin = jnp.where(better, t, tmin)
                cmin = jnp.where(better, col, cmin)
                xmin = jnp.where(better, x, xmin)
            return tmin, cmin, xmin

        return lax.fori_loop(0, size // (unroll * SC_LANES), step, carry)

    carry = (
        jnp.full((SC_LANES,), jnp.inf, jnp.float32),
        jnp.zeros((SC_LANES,), jnp.int32),
        jnp.zeros((SC_LANES,), jnp.float32),
    )
    h_cur = start(0)
    for j in range(len(SC_SIZES)):
        h_next = start(j + 1) if j + 1 < len(SC_SIZES) else None
        h_cur.wait()
        carry = process(chunk_buf(j), SC_SIZES[j], C0 + j * SC_CH, carry)
        h_cur = h_next

    _, cmin, xmin = carry
    col_v[...] = cmin
    xv_v[...] = xmin
    pltpu.sync_copy(col_v, col_out.at[row])
    pltpu.sync_copy(xv_v, x_out.at[row])


@functools.cache
def _sc_tail_kernel():
    return pl.kernel(
        _sc_body,
        out_type=(
            jax.ShapeDtypeStruct((B, SC_LANES), jnp.int32),
            jax.ShapeDtypeStruct((B, SC_LANES), jnp.float32),
        ),
        mesh=plsc.VectorSubcoreMesh(core_axis_name="c", subcore_axis_name="s"),
        scratch_types=[
            pltpu.VMEM((SC_CH,), jnp.float32),
            pltpu.VMEM((SC_CH,), jnp.float32),
            pltpu.VMEM((SC_SIZES[-1],), jnp.float32),
            pltpu.VMEM((SC_LANES,), jnp.int32),
            pltpu.VMEM((SC_LANES,), jnp.float32),
            pltpu.SemaphoreType.DMA,
            pltpu.SemaphoreType.DMA,
        ],
    )


def _merge_kernel(val_ref, idx_ref, sccol_ref, scx_ref, o_ref):
    col = sccol_ref[...]
    x = scx_ref[...]
    row = jax.lax.broadcasted_iota(jnp.uint32, (B, SC_LANES), 0)
    flat = row * jnp.uint32(V) + col.astype(jnp.uint32)
    s = _score(_threefry_bits(flat), x)
    m = jnp.max(s, axis=1, keepdims=True)
    cand = jnp.where(s == m, col, jnp.int32(2**31 - 1))
    sc_col = jnp.min(cand, axis=1)
    sc_val = m[:, 0]
    take_sc = sc_val > val_ref[0, :]
    o_ref[0, :] = jnp.where(take_sc, sc_col, idx_ref[0, :])


def _merge(tc_val, tc_idx, sc_col, sc_x):
    return pl.pallas_call(
        _merge_kernel,
        out_shape=jax.ShapeDtypeStruct((1, B), jnp.int32),
    )(tc_val, tc_idx, sc_col, sc_x)


@jax.jit
def kernel(logits):
    sc_col, sc_x = _sc_tail_kernel()(logits)
    tc_val, tc_idx = _tc_partial(logits)
    out = _merge(tc_val, tc_idx, sc_col, sc_x)
    return out[0]
